# baseline (speedup 1.0000x reference)
import jax
import jax.numpy as jnp
from jax.experimental import pallas as pl


def _body(xu_ref, xi_ref, uh_ref, av_ref, cu_ref, cir_ref, s_ref, t_ref, out_ref):
    coef_user = jnp.dot(uh_ref[...], cu_ref[...], preferred_element_type=jnp.float32)
    c_r = jnp.dot(coef_user, t_ref[...], preferred_element_type=jnp.float32)
    acc = xu_ref[...] * c_r + xi_ref[...] * cir_ref[...]
    out = jnp.dot(acc, s_ref[...], preferred_element_type=jnp.float32)
    out_ref[...] = jnp.where(av_ref[...], out, jnp.float32(-1e20))


def kernel(x_u, x_i, availability, user_onehot, coef_u, coef_i):
    B, I, P = x_u.shape
    U = coef_u.shape[0]
    IP = I * P
    xu2 = x_u.reshape(B, IP)
    xi2 = x_i.reshape(B, IP)
    uh2 = user_onehot.reshape(B, U)
    ci_r = coef_i.reshape(1, IP)
    ii = jnp.arange(IP, dtype=jnp.int32)
    S = (ii[:, None] // P == jnp.arange(I, dtype=jnp.int32)[None, :]).astype(jnp.float32)
    T = (jnp.arange(P, dtype=jnp.int32)[:, None] == (ii[None, :] % P)).astype(jnp.float32)

    Bb = 256
    out = pl.pallas_call(
        _body,
        grid=(B // Bb,),
        in_specs=[
            pl.BlockSpec((Bb, IP), lambda i: (i, 0)),
            pl.BlockSpec((Bb, IP), lambda i: (i, 0)),
            pl.BlockSpec((Bb, U), lambda i: (i, 0)),
            pl.BlockSpec((Bb, I), lambda i: (i, 0)),
            pl.BlockSpec((U, P), lambda i: (0, 0)),
            pl.BlockSpec((1, IP), lambda i: (0, 0)),
            pl.BlockSpec((IP, I), lambda i: (0, 0)),
            pl.BlockSpec((P, IP), lambda i: (0, 0)),
        ],
        out_specs=pl.BlockSpec((Bb, I), lambda i: (i, 0)),
        out_shape=jax.ShapeDtypeStruct((B, I), jnp.float32),
    )(xu2, xi2, uh2, availability, coef_u, ci_r, S, T)
    return out


# speedup vs baseline: 4.0526x; 4.0526x over previous
import jax
import jax.numpy as jnp
from jax.experimental import pallas as pl


def _body(uh_ref, xu_ref, xi_ref, av_ref, cut_ref, cie_ref, st_ref, out_ref):
    I, P, Bn = xu_ref.shape
    cu = jnp.dot(cut_ref[...], uh_ref[...], preferred_element_type=jnp.float32)
    y = xu_ref[...] * cu[None, :, :] + xi_ref[...] * cie_ref[...]
    out = jnp.dot(st_ref[...], y.reshape(I * P, Bn),
                  preferred_element_type=jnp.float32)
    out_ref[...] = jnp.where(av_ref[...], out, jnp.float32(-1e20))


def kernel(x_u, x_i, availability, user_onehot, coef_u, coef_i):
    B, I, P = x_u.shape
    U = coef_u.shape[0]
    IP = I * P

    xu_t = jnp.transpose(x_u, (1, 2, 0))
    xi_t = jnp.transpose(x_i, (1, 2, 0))
    uh_t = jnp.transpose(user_onehot, (1, 2, 0)).reshape(U, B)
    av_t = availability.T
    cu_t = coef_u.T
    ci_e = coef_i[:, :, None]
    jj = jnp.arange(IP, dtype=jnp.int32)
    s_t = (jj[None, :] // P == jnp.arange(I, dtype=jnp.int32)[:, None]).astype(jnp.float32)

    Bn = 512
    out_t = pl.pallas_call(
        _body,
        grid=(B // Bn,),
        in_specs=[
            pl.BlockSpec((U, Bn), lambda i: (0, i)),
            pl.BlockSpec((I, P, Bn), lambda i: (0, 0, i)),
            pl.BlockSpec((I, P, Bn), lambda i: (0, 0, i)),
            pl.BlockSpec((I, Bn), lambda i: (0, i)),
            pl.BlockSpec((P, U), lambda i: (0, 0)),
            pl.BlockSpec((I, P, 1), lambda i: (0, 0, 0)),
            pl.BlockSpec((I, IP), lambda i: (0, 0)),
        ],
        out_specs=pl.BlockSpec((I, Bn), lambda i: (0, i)),
        out_shape=jax.ShapeDtypeStruct((I, B), jnp.float32),
    )(uh_t, xu_t, xi_t, av_t, cu_t, ci_e, s_t)
    return out_t.T


# speedup vs baseline: 4.1919x; 1.0344x over previous
import jax
import jax.numpy as jnp
from jax.experimental import pallas as pl


def _body(uh_ref, xu_ref, xi_ref, av_ref, cut_ref, cie_ref, st_ref, out_ref):
    I, P, Bn = xu_ref.shape
    cu = jnp.dot(cut_ref[...], uh_ref[...], preferred_element_type=jnp.float32)
    y = xu_ref[...] * cu[None, :, :] + xi_ref[...] * cie_ref[...]
    out = jnp.dot(st_ref[...], y.reshape(I * P, Bn),
                  preferred_element_type=jnp.float32)
    out_ref[...] = jnp.where(av_ref[...] != 0, out, jnp.float32(-1e20))


def kernel(x_u, x_i, availability, user_onehot, coef_u, coef_i):
    B, I, P = x_u.shape
    U = coef_u.shape[0]
    IP = I * P

    xu_t = jnp.transpose(x_u, (1, 2, 0))
    xi_t = jnp.transpose(x_i, (1, 2, 0))
    uh_t = jnp.transpose(user_onehot, (1, 2, 0)).reshape(U, B)
    av_t = availability.T.astype(jnp.int8)
    cu_t = coef_u.T
    ci_e = coef_i[:, :, None]
    jj = jnp.arange(IP, dtype=jnp.int32)
    s_t = (jj[None, :] // P == jnp.arange(I, dtype=jnp.int32)[:, None]).astype(jnp.float32)

    Bn = 512
    out_t = pl.pallas_call(
        _body,
        grid=(B // Bn,),
        in_specs=[
            pl.BlockSpec((U, Bn), lambda i: (0, i)),
            pl.BlockSpec((I, P, Bn), lambda i: (0, 0, i)),
            pl.BlockSpec((I, P, Bn), lambda i: (0, 0, i)),
            pl.BlockSpec((I, Bn), lambda i: (0, i)),
            pl.BlockSpec((P, U), lambda i: (0, 0)),
            pl.BlockSpec((I, P, 1), lambda i: (0, 0, 0)),
            pl.BlockSpec((I, IP), lambda i: (0, 0)),
        ],
        out_specs=pl.BlockSpec((I, Bn), lambda i: (0, i)),
        out_shape=jax.ShapeDtypeStruct((I, B), jnp.float32),
    )(uh_t, xu_t, xi_t, av_t, cu_t, ci_e, s_t)
    return out_t.T
